# Initial kernel scaffold; baseline (speedup 1.0000x reference)
import jax
import jax.numpy as jnp
from jax.experimental import pallas as pl


def kernel(input_ids, word_embeddings):
    raise NotImplementedError("write your pallas kernel here")


# speedup vs baseline: 1.8188x; 1.8188x over previous
import functools

import jax
import jax.numpy as jnp
from jax import lax
from jax.experimental import pallas as pl
from jax.experimental.pallas import tpu as pltpu
from jax.experimental.pallas import tpu_sc as plsc

VOCAB = 100
HIDDEN = 128
BATCH = 4096
HIST = 50

_B = BATCH * HIST
_NC, _NS = 2, 16
_NW = _NC * _NS
_BPW = _B // _NW
_CHUNK = 128
_NCHUNK = _BPW // _CHUNK


def _make_gather():
    mesh = plsc.VectorSubcoreMesh(core_axis_name="c", subcore_axis_name="s")

    @functools.partial(
        pl.kernel,
        out_type=jax.ShapeDtypeStruct((_B, HIDDEN), jnp.float32),
        mesh=mesh,
        scratch_types=[
            pltpu.VMEM((_NCHUNK, _CHUNK), jnp.int32),
            pltpu.VMEM((_CHUNK, HIDDEN), jnp.float32),
            pltpu.SemaphoreType.DMA,
        ],
    )
    def gather_kernel(idx_hbm, table_hbm, out_hbm, idx_v, rows_v, gsem):
        wid = lax.axis_index("s") * _NC + lax.axis_index("c")
        base = wid * _BPW

        pltpu.sync_copy(idx_hbm.at[wid], idx_v)

        def chunk_body(j):
            pltpu.async_copy(table_hbm.at[idx_v.at[j]], rows_v, gsem).wait()
            pltpu.sync_copy(
                rows_v, out_hbm.at[pl.ds(base + j * _CHUNK, _CHUNK)]
            )

        pl.loop(0, _NCHUNK)(chunk_body)

    return gather_kernel


_gather = _make_gather()


def kernel(input_ids, word_embeddings):
    ids = input_ids.reshape(_NW, _NCHUNK, _CHUNK).astype(jnp.int32)
    out = _gather(ids, word_embeddings)
    return out.reshape(BATCH, HIST, HIDDEN)


# speedup vs baseline: 1.8292x; 1.0058x over previous
import functools

import jax
import jax.numpy as jnp
from jax import lax
from jax.experimental import pallas as pl
from jax.experimental.pallas import tpu as pltpu
from jax.experimental.pallas import tpu_sc as plsc

VOCAB = 100
HIDDEN = 128
BATCH = 4096
HIST = 50

_B = BATCH * HIST
_NC, _NS = 2, 16
_NW = _NC * _NS
_BPW = _B // _NW
_CHUNK = 128
_NCHUNK = _BPW // _CHUNK
_K = 2
_GROUP = _K * _CHUNK
_NGROUP = _NCHUNK // _K


def _make_gather():
    mesh = plsc.VectorSubcoreMesh(core_axis_name="c", subcore_axis_name="s")

    @functools.partial(
        pl.kernel,
        out_type=jax.ShapeDtypeStruct((_B, HIDDEN), jnp.float32),
        mesh=mesh,
        scratch_types=[
            pltpu.VMEM((_NCHUNK, _CHUNK), jnp.int32),
            pltpu.VMEM((_GROUP, HIDDEN), jnp.float32),
            pltpu.VMEM((_GROUP, HIDDEN), jnp.float32),
            pltpu.SemaphoreType.DMA,
            pltpu.SemaphoreType.DMA,
            pltpu.SemaphoreType.DMA,
            pltpu.SemaphoreType.DMA,
        ],
    )
    def gather_kernel(idx_hbm, table_hbm, out_hbm,
                      idx_v, buf_a, buf_b, gs_a, gs_b, ss_a, ss_b):
        wid = lax.axis_index("s") * _NC + lax.axis_index("c")
        base = wid * _BPW

        pltpu.sync_copy(idx_hbm.at[wid], idx_v)

        def start_gathers(g, buf, sem):
            for b in range(_K):
                pltpu.async_copy(
                    table_hbm.at[idx_v.at[g * _K + b]],
                    buf.at[pl.ds(b * _CHUNK, _CHUNK)],
                    sem,
                )

        def wait_gathers(g, buf, sem):
            for b in range(_K):
                pltpu.make_async_copy(
                    table_hbm.at[idx_v.at[g * _K + b]],
                    buf.at[pl.ds(b * _CHUNK, _CHUNK)],
                    sem,
                ).wait()

        def start_store(g, buf, sem):
            pltpu.async_copy(
                buf, out_hbm.at[pl.ds(base + g * _GROUP, _GROUP)], sem
            )

        def wait_store(buf, sem):
            pltpu.make_async_copy(
                buf, out_hbm.at[pl.ds(base, _GROUP)], sem
            ).wait()

        start_gathers(0, buf_a, gs_a)
        start_gathers(1, buf_b, gs_b)

        def group_pair(i):
            g = 2 * i
            wait_gathers(g, buf_a, gs_a)
            start_store(g, buf_a, ss_a)
            wait_gathers(g + 1, buf_b, gs_b)
            start_store(g + 1, buf_b, ss_b)
            wait_store(buf_a, ss_a)
            start_gathers(g + 2, buf_a, gs_a)
            wait_store(buf_b, ss_b)
            start_gathers(g + 3, buf_b, gs_b)

        pl.loop(0, (_NGROUP - 3) // 2)(group_pair)

        g = _NGROUP - 3
        wait_gathers(g, buf_a, gs_a)
        start_store(g, buf_a, ss_a)
        wait_gathers(g + 1, buf_b, gs_b)
        start_store(g + 1, buf_b, ss_b)
        wait_store(buf_a, ss_a)
        start_gathers(g + 2, buf_a, gs_a)
        wait_gathers(g + 2, buf_a, gs_a)
        start_store(g + 2, buf_a, ss_a)
        wait_store(buf_a, ss_a)
        wait_store(buf_b, ss_b)

    return gather_kernel


_gather = _make_gather()


def kernel(input_ids, word_embeddings):
    ids = input_ids.reshape(_NW, _NCHUNK, _CHUNK).astype(jnp.int32)
    out = _gather(ids, word_embeddings)
    return out.reshape(BATCH, HIST, HIDDEN)


# speedup vs baseline: 2.4537x; 1.3414x over previous
import functools

import jax
import jax.numpy as jnp
from jax import lax
from jax.experimental import pallas as pl
from jax.experimental.pallas import tpu as pltpu
from jax.experimental.pallas import tpu_sc as plsc

VOCAB = 100
HIDDEN = 128
BATCH = 4096
HIST = 50

_NC, _NS = 2, 16
_NW = _NC * _NS
_ROWS_PW = BATCH // _NW
_GB = 8
_NGROUP = _ROWS_PW // _GB


def _make_gather():
    mesh = plsc.VectorSubcoreMesh(core_axis_name="c", subcore_axis_name="s")

    @functools.partial(
        pl.kernel,
        out_type=jax.ShapeDtypeStruct((BATCH, HIST, HIDDEN), jnp.float32),
        mesh=mesh,
        scratch_types=[
            pltpu.VMEM((_ROWS_PW, HIST), jnp.int32),
            pltpu.VMEM((_GB, HIST, HIDDEN), jnp.float32),
            pltpu.VMEM((_GB, HIST, HIDDEN), jnp.float32),
            pltpu.SemaphoreType.DMA,
            pltpu.SemaphoreType.DMA,
            pltpu.SemaphoreType.DMA,
            pltpu.SemaphoreType.DMA,
        ],
    )
    def gather_kernel(idx_hbm, table_hbm, out_hbm,
                      idx_v, buf_a, buf_b, gs_a, gs_b, ss_a, ss_b):
        wid = lax.axis_index("s") * _NC + lax.axis_index("c")
        base = wid * _ROWS_PW

        pltpu.sync_copy(idx_hbm.at[pl.ds(base, _ROWS_PW)], idx_v)

        def start_gathers(g, buf, sem):
            for b in range(_GB):
                pltpu.async_copy(
                    table_hbm.at[idx_v.at[g * _GB + b]], buf.at[b], sem
                )

        def wait_gathers(g, buf, sem):
            for b in range(_GB):
                pltpu.make_async_copy(
                    table_hbm.at[idx_v.at[g * _GB + b]], buf.at[b], sem
                ).wait()

        def start_store(g, buf, sem):
            pltpu.async_copy(
                buf, out_hbm.at[pl.ds(base + g * _GB, _GB)], sem
            )

        def wait_store(buf, sem):
            pltpu.make_async_copy(
                buf, out_hbm.at[pl.ds(base, _GB)], sem
            ).wait()

        start_gathers(0, buf_a, gs_a)
        start_gathers(1, buf_b, gs_b)

        def group_pair(i):
            g = 2 * i
            wait_gathers(g, buf_a, gs_a)
            start_store(g, buf_a, ss_a)
            wait_gathers(g + 1, buf_b, gs_b)
            start_store(g + 1, buf_b, ss_b)
            wait_store(buf_a, ss_a)
            start_gathers(g + 2, buf_a, gs_a)
            wait_store(buf_b, ss_b)
            start_gathers(g + 3, buf_b, gs_b)

        pl.loop(0, (_NGROUP - 2) // 2)(group_pair)

        g = _NGROUP - 2
        wait_gathers(g, buf_a, gs_a)
        start_store(g, buf_a, ss_a)
        wait_gathers(g + 1, buf_b, gs_b)
        start_store(g + 1, buf_b, ss_b)
        wait_store(buf_a, ss_a)
        wait_store(buf_b, ss_b)

    return gather_kernel


_gather = _make_gather()


def kernel(input_ids, word_embeddings):
    return _gather(input_ids.astype(jnp.int32), word_embeddings)


# speedup vs baseline: 6.3072x; 2.5705x over previous
import functools

import jax
import jax.numpy as jnp
from jax import lax
from jax.experimental import pallas as pl
from jax.experimental.pallas import tpu as pltpu
from jax.experimental.pallas import tpu_sc as plsc

VOCAB = 100
HIDDEN = 128
BATCH = 4096
HIST = 50

_NC, _NS = 2, 16
_NW = _NC * _NS
_ROWS_PW = BATCH // _NW
_GB = 4
_NGROUP = _ROWS_PW // _GB


def _make_gather():
    mesh = plsc.VectorSubcoreMesh(core_axis_name="c", subcore_axis_name="s")

    @functools.partial(
        pl.kernel,
        out_type=jax.ShapeDtypeStruct((BATCH, HIST, HIDDEN), jnp.float32),
        mesh=mesh,
        scratch_types=[
            pltpu.VMEM_SHARED((VOCAB, HIDDEN), jnp.float32),
            pltpu.VMEM((_ROWS_PW, HIST), jnp.int32),
            pltpu.VMEM((_GB, HIST, HIDDEN), jnp.float32),
            pltpu.VMEM((_GB, HIST, HIDDEN), jnp.float32),
            pltpu.SemaphoreType.DMA,
            pltpu.SemaphoreType.DMA,
            pltpu.SemaphoreType.DMA,
            pltpu.SemaphoreType.DMA,
        ],
    )
    def gather_kernel(idx_hbm, table_hbm, out_hbm,
                      table_sh, idx_v, buf_a, buf_b, gs_a, gs_b, ss_a, ss_b):
        sid = lax.axis_index("s")
        wid = sid * _NC + lax.axis_index("c")
        base = wid * _ROWS_PW

        @pl.when(sid == 0)
        def _():
            pltpu.sync_copy(table_hbm, table_sh)

        pltpu.sync_copy(idx_hbm.at[pl.ds(base, _ROWS_PW)], idx_v)
        plsc.subcore_barrier()

        def start_gathers(g, buf, sem):
            for b in range(_GB):
                pltpu.async_copy(
                    table_sh.at[idx_v.at[g * _GB + b]], buf.at[b], sem
                )

        def wait_gathers(g, buf, sem):
            for b in range(_GB):
                pltpu.make_async_copy(
                    table_sh.at[idx_v.at[g * _GB + b]], buf.at[b], sem
                ).wait()

        def start_store(g, buf, sem):
            pltpu.async_copy(
                buf, out_hbm.at[pl.ds(base + g * _GB, _GB)], sem
            )

        def wait_store(buf, sem):
            pltpu.make_async_copy(
                buf, out_hbm.at[pl.ds(base, _GB)], sem
            ).wait()

        start_gathers(0, buf_a, gs_a)
        start_gathers(1, buf_b, gs_b)

        def group_pair(i):
            g = 2 * i
            wait_gathers(g, buf_a, gs_a)
            start_store(g, buf_a, ss_a)
            wait_gathers(g + 1, buf_b, gs_b)
            start_store(g + 1, buf_b, ss_b)
            wait_store(buf_a, ss_a)
            start_gathers(g + 2, buf_a, gs_a)
            wait_store(buf_b, ss_b)
            start_gathers(g + 3, buf_b, gs_b)

        pl.loop(0, (_NGROUP - 2) // 2)(group_pair)

        g = _NGROUP - 2
        wait_gathers(g, buf_a, gs_a)
        start_store(g, buf_a, ss_a)
        wait_gathers(g + 1, buf_b, gs_b)
        start_store(g + 1, buf_b, ss_b)
        wait_store(buf_a, ss_a)
        wait_store(buf_b, ss_b)

    return gather_kernel


_gather = _make_gather()


def kernel(input_ids, word_embeddings):
    return _gather(input_ids.astype(jnp.int32), word_embeddings)


# speedup vs baseline: 11.7700x; 1.8661x over previous
import functools

import jax
import jax.numpy as jnp
from jax import lax
from jax.experimental import pallas as pl
from jax.experimental.pallas import tpu as pltpu
from jax.experimental.pallas import tpu_sc as plsc

VOCAB = 100
HIDDEN = 128
BATCH = 4096
HIST = 50

_NC, _NS = 2, 16
_NW = _NC * _NS
_COLS_PW = BATCH // _NW
_TG = 2
_NGROUP = HIST // _TG


def _make_gather():
    mesh = plsc.VectorSubcoreMesh(core_axis_name="c", subcore_axis_name="s")

    @functools.partial(
        pl.kernel,
        out_type=jax.ShapeDtypeStruct((HIST, BATCH, HIDDEN), jnp.float32),
        mesh=mesh,
        scratch_types=[
            pltpu.VMEM_SHARED((VOCAB, HIDDEN), jnp.float32),
            pltpu.VMEM((HIST, _COLS_PW), jnp.int32),
            pltpu.VMEM((_TG, _COLS_PW, HIDDEN), jnp.float32),
            pltpu.VMEM((_TG, _COLS_PW, HIDDEN), jnp.float32),
            pltpu.SemaphoreType.DMA,
            pltpu.SemaphoreType.DMA,
            pltpu.SemaphoreType.DMA,
            pltpu.SemaphoreType.DMA,
        ],
    )
    def gather_kernel(idx_hbm, table_hbm, out_hbm,
                      table_sh, idx_v, buf_a, buf_b, gs_a, gs_b, ss_a, ss_b):
        sid = lax.axis_index("s")
        wid = sid * _NC + lax.axis_index("c")
        base = wid * _COLS_PW

        @pl.when(sid == 0)
        def _():
            pltpu.sync_copy(table_hbm, table_sh)

        pltpu.sync_copy(idx_hbm.at[:, pl.ds(base, _COLS_PW)], idx_v)
        plsc.subcore_barrier()

        def start_gathers(g, buf, sem):
            for b in range(_TG):
                pltpu.async_copy(
                    table_sh.at[idx_v.at[g * _TG + b]], buf.at[b], sem
                )

        def wait_gathers(g, buf, sem):
            for b in range(_TG):
                pltpu.make_async_copy(
                    table_sh.at[idx_v.at[g * _TG + b]], buf.at[b], sem
                ).wait()

        def start_store(g, buf, sem):
            pltpu.async_copy(
                buf, out_hbm.at[pl.ds(g * _TG, _TG), pl.ds(base, _COLS_PW)],
                sem,
            )

        def wait_store(buf, sem):
            pltpu.make_async_copy(
                buf, out_hbm.at[pl.ds(0, _TG), pl.ds(base, _COLS_PW)], sem
            ).wait()

        start_gathers(0, buf_a, gs_a)
        start_gathers(1, buf_b, gs_b)

        def group_pair(i):
            g = 2 * i
            wait_gathers(g, buf_a, gs_a)
            start_store(g, buf_a, ss_a)
            wait_gathers(g + 1, buf_b, gs_b)
            start_store(g + 1, buf_b, ss_b)
            wait_store(buf_a, ss_a)
            start_gathers(g + 2, buf_a, gs_a)
            wait_store(buf_b, ss_b)
            start_gathers(g + 3, buf_b, gs_b)

        pl.loop(0, (_NGROUP - 3) // 2)(group_pair)

        g = _NGROUP - 3
        wait_gathers(g, buf_a, gs_a)
        start_store(g, buf_a, ss_a)
        wait_gathers(g + 1, buf_b, gs_b)
        start_store(g + 1, buf_b, ss_b)
        wait_store(buf_a, ss_a)
        start_gathers(g + 2, buf_a, gs_a)
        wait_gathers(g + 2, buf_a, gs_a)
        start_store(g + 2, buf_a, ss_a)
        wait_store(buf_a, ss_a)
        wait_store(buf_b, ss_b)

    return gather_kernel


_gather = _make_gather()


def kernel(input_ids, word_embeddings):
    ids_t = input_ids.astype(jnp.int32).T
    out = _gather(ids_t, word_embeddings)
    return jnp.transpose(out, (1, 0, 2))


# speedup vs baseline: 11.7839x; 1.0012x over previous
import functools

import jax
import jax.numpy as jnp
from jax import lax
from jax.experimental import pallas as pl
from jax.experimental.pallas import tpu as pltpu
from jax.experimental.pallas import tpu_sc as plsc

VOCAB = 100
HIDDEN = 128
BATCH = 4096
HIST = 50

_NC, _NS = 2, 16
_NW = _NC * _NS
_COLS_PW = BATCH // _NW
_TG = 3
_NGROUP = -(-HIST // _TG)
_LAST = HIST - (_NGROUP - 1) * _TG


def _make_gather():
    mesh = plsc.VectorSubcoreMesh(core_axis_name="c", subcore_axis_name="s")

    @functools.partial(
        pl.kernel,
        out_type=jax.ShapeDtypeStruct((HIST, BATCH, HIDDEN), jnp.float32),
        mesh=mesh,
        scratch_types=[
            pltpu.VMEM_SHARED((VOCAB, HIDDEN), jnp.float32),
            pltpu.VMEM((HIST, _COLS_PW), jnp.int32),
            pltpu.VMEM((_TG, _COLS_PW, HIDDEN), jnp.float32),
            pltpu.VMEM((_TG, _COLS_PW, HIDDEN), jnp.float32),
            pltpu.SemaphoreType.DMA,
            pltpu.SemaphoreType.DMA,
            pltpu.SemaphoreType.DMA,
            pltpu.SemaphoreType.DMA,
        ],
    )
    def gather_kernel(idx_hbm, table_hbm, out_hbm,
                      table_sh, idx_v, buf_a, buf_b, gs_a, gs_b, ss_a, ss_b):
        sid = lax.axis_index("s")
        wid = sid * _NC + lax.axis_index("c")
        base = wid * _COLS_PW

        @pl.when(sid == 0)
        def _():
            pltpu.sync_copy(table_hbm, table_sh)

        pltpu.sync_copy(idx_hbm.at[:, pl.ds(base, _COLS_PW)], idx_v)
        plsc.subcore_barrier()

        def start_gathers(g, buf, sem, n=_TG):
            for b in range(n):
                pltpu.async_copy(
                    table_sh.at[idx_v.at[g * _TG + b]], buf.at[b], sem
                )

        def wait_gathers(g, buf, sem, n=_TG):
            for b in range(n):
                pltpu.make_async_copy(
                    table_sh.at[idx_v.at[g * _TG + b]], buf.at[b], sem
                ).wait()

        def start_store(g, buf, sem, n=_TG):
            pltpu.async_copy(
                buf.at[pl.ds(0, n)],
                out_hbm.at[pl.ds(g * _TG, n), pl.ds(base, _COLS_PW)],
                sem,
            )

        def wait_store(buf, sem, n=_TG):
            pltpu.make_async_copy(
                buf.at[pl.ds(0, n)],
                out_hbm.at[pl.ds(0, n), pl.ds(base, _COLS_PW)], sem
            ).wait()

        start_gathers(0, buf_a, gs_a)
        start_gathers(1, buf_b, gs_b)

        def group_pair(i):
            g = 2 * i
            wait_gathers(g, buf_a, gs_a)
            start_store(g, buf_a, ss_a)
            wait_gathers(g + 1, buf_b, gs_b)
            start_store(g + 1, buf_b, ss_b)
            wait_store(buf_a, ss_a)
            start_gathers(g + 2, buf_a, gs_a)
            wait_store(buf_b, ss_b)
            start_gathers(g + 3, buf_b, gs_b)

        pl.loop(0, (_NGROUP - 3) // 2)(group_pair)

        g = _NGROUP - 3
        wait_gathers(g, buf_a, gs_a)
        start_store(g, buf_a, ss_a)
        wait_gathers(g + 1, buf_b, gs_b)
        start_store(g + 1, buf_b, ss_b)
        wait_store(buf_a, ss_a)
        start_gathers(g + 2, buf_a, gs_a, n=_LAST)
        wait_gathers(g + 2, buf_a, gs_a, n=_LAST)
        start_store(g + 2, buf_a, ss_a, n=_LAST)
        wait_store(buf_a, ss_a, n=_LAST)
        wait_store(buf_b, ss_b)

    return gather_kernel


_gather = _make_gather()


def kernel(input_ids, word_embeddings):
    ids_t = input_ids.astype(jnp.int32).T
    out = _gather(ids_t, word_embeddings)
    return jnp.transpose(out, (1, 0, 2))


# speedup vs baseline: 14.7146x; 1.2487x over previous
import functools

import jax
import jax.numpy as jnp
from jax import lax
from jax.experimental import pallas as pl
from jax.experimental.pallas import tpu as pltpu
from jax.experimental.pallas import tpu_sc as plsc

VOCAB = 100
HIDDEN = 128
BATCH = 4096
HIST = 50

_NC, _NS = 2, 16
_NW = _NC * _NS
_COLS_PW = BATCH // _NW
_TG = 2
_NGROUP = HIST // _TG


def _make_gather():
    mesh = plsc.VectorSubcoreMesh(core_axis_name="c", subcore_axis_name="s")

    @functools.partial(
        pl.kernel,
        out_type=jax.ShapeDtypeStruct((HIST, BATCH, HIDDEN), jnp.float32),
        mesh=mesh,
        scratch_types=[
            pltpu.VMEM_SHARED((VOCAB, HIDDEN), jnp.float32),
            pltpu.VMEM((HIST, _COLS_PW), jnp.int32),
            pltpu.VMEM((_TG, _COLS_PW, HIDDEN), jnp.float32),
            pltpu.VMEM((_TG, _COLS_PW, HIDDEN), jnp.float32),
            pltpu.VMEM((_TG, _COLS_PW, HIDDEN), jnp.float32),
            pltpu.SemaphoreType.DMA,
            pltpu.SemaphoreType.DMA,
            pltpu.SemaphoreType.DMA,
            pltpu.SemaphoreType.DMA,
            pltpu.SemaphoreType.DMA,
            pltpu.SemaphoreType.DMA,
        ],
    )
    def gather_kernel(idx_hbm, table_hbm, out_hbm,
                      table_sh, idx_v, buf_a, buf_b, buf_c,
                      gs_a, gs_b, gs_c, ss_a, ss_b, ss_c):
        sid = lax.axis_index("s")
        wid = sid * _NC + lax.axis_index("c")
        base = wid * _COLS_PW

        @pl.when(sid == 0)
        def _():
            pltpu.sync_copy(table_hbm, table_sh)

        pltpu.sync_copy(idx_hbm.at[:, pl.ds(base, _COLS_PW)], idx_v)
        plsc.subcore_barrier()

        def start_gathers(g, buf, sem, n=_TG):
            for b in range(n):
                pltpu.async_copy(
                    table_sh.at[idx_v.at[g * _TG + b]], buf.at[b], sem
                )

        def wait_gathers(g, buf, sem, n=_TG):
            for b in range(n):
                pltpu.make_async_copy(
                    table_sh.at[idx_v.at[g * _TG + b]], buf.at[b], sem
                ).wait()

        def start_store(g, buf, sem, n=_TG):
            pltpu.async_copy(
                buf.at[pl.ds(0, n)],
                out_hbm.at[pl.ds(g * _TG, n), pl.ds(base, _COLS_PW)],
                sem,
            )

        def wait_store(buf, sem, n=_TG):
            pltpu.make_async_copy(
                buf.at[pl.ds(0, n)],
                out_hbm.at[pl.ds(0, n), pl.ds(base, _COLS_PW)], sem
            ).wait()

        rings = ((buf_a, gs_a, ss_a), (buf_b, gs_b, ss_b), (buf_c, gs_c, ss_c))

        for k, (buf, gs, _) in enumerate(rings):
            start_gathers(k, buf, gs)

        def group_triple(i):
            g = 3 * i
            for k, (buf, gs, ss) in enumerate(rings):
                wait_gathers(g + k, buf, gs)
                start_store(g + k, buf, ss)
            for k, (buf, gs, ss) in enumerate(rings):
                wait_store(buf, ss)
                start_gathers(g + 3 + k, buf, gs)

        pl.loop(0, (_NGROUP - 4) // 3)(group_triple)

        g = _NGROUP - 4
        for k, (buf, gs, ss) in enumerate(rings):
            wait_gathers(g + k, buf, gs)
            start_store(g + k, buf, ss)
        wait_store(buf_a, ss_a)
        start_gathers(g + 3, buf_a, gs_a)
        wait_gathers(g + 3, buf_a, gs_a)
        start_store(g + 3, buf_a, ss_a)
        wait_store(buf_a, ss_a)
        wait_store(buf_b, ss_b)
        wait_store(buf_c, ss_c)

    return gather_kernel


_gather = _make_gather()


def kernel(input_ids, word_embeddings):
    ids_t = input_ids.astype(jnp.int32).T
    out = _gather(ids_t, word_embeddings)
    return jnp.transpose(out, (1, 0, 2))


# speedup vs baseline: 15.6924x; 1.0665x over previous
import functools

import jax
import jax.numpy as jnp
from jax import lax
from jax.experimental import pallas as pl
from jax.experimental.pallas import tpu as pltpu
from jax.experimental.pallas import tpu_sc as plsc

VOCAB = 100
HIDDEN = 128
BATCH = 4096
HIST = 50

_NC, _NS = 2, 16
_NW = _NC * _NS
_COLS_PW = BATCH // _NW
_TG = 1
_NGROUP = HIST // _TG
_RING = 6


def _make_gather():
    mesh = plsc.VectorSubcoreMesh(core_axis_name="c", subcore_axis_name="s")

    @functools.partial(
        pl.kernel,
        out_type=jax.ShapeDtypeStruct((HIST, BATCH, HIDDEN), jnp.float32),
        mesh=mesh,
        scratch_types=[
            pltpu.VMEM_SHARED((VOCAB, HIDDEN), jnp.float32),
            pltpu.VMEM((HIST, _COLS_PW), jnp.int32),
            *([pltpu.VMEM((_TG, _COLS_PW, HIDDEN), jnp.float32)] * _RING),
            *([pltpu.SemaphoreType.DMA] * (2 * _RING)),
        ],
    )
    def gather_kernel(idx_hbm, table_hbm, out_hbm, table_sh, idx_v, *rest):
        bufs = rest[:_RING]
        gsems = rest[_RING:2 * _RING]
        ssems = rest[2 * _RING:]
        sid = lax.axis_index("s")
        wid = sid * _NC + lax.axis_index("c")
        base = wid * _COLS_PW

        @pl.when(sid == 0)
        def _():
            pltpu.sync_copy(table_hbm, table_sh)

        pltpu.sync_copy(idx_hbm.at[:, pl.ds(base, _COLS_PW)], idx_v)
        plsc.subcore_barrier()

        def start_gathers(g, buf, sem, n=_TG):
            for b in range(n):
                pltpu.async_copy(
                    table_sh.at[idx_v.at[g * _TG + b]], buf.at[b], sem
                )

        def wait_gathers(g, buf, sem, n=_TG):
            for b in range(n):
                pltpu.make_async_copy(
                    table_sh.at[idx_v.at[g * _TG + b]], buf.at[b], sem
                ).wait()

        def start_store(g, buf, sem, n=_TG):
            pltpu.async_copy(
                buf.at[pl.ds(0, n)],
                out_hbm.at[pl.ds(g * _TG, n), pl.ds(base, _COLS_PW)],
                sem,
            )

        def wait_store(buf, sem, n=_TG):
            pltpu.make_async_copy(
                buf.at[pl.ds(0, n)],
                out_hbm.at[pl.ds(0, n), pl.ds(base, _COLS_PW)], sem
            ).wait()

        rings = tuple(zip(bufs, gsems, ssems))

        for k, (buf, gs, _) in enumerate(rings):
            start_gathers(k, buf, gs)

        def group_round(i):
            g = _RING * i
            for k, (buf, gs, ss) in enumerate(rings):
                wait_gathers(g + k, buf, gs)
                start_store(g + k, buf, ss)
            for k, (buf, gs, ss) in enumerate(rings):
                wait_store(buf, ss)
                start_gathers(g + _RING + k, buf, gs)

        _NFULL = (_NGROUP - _RING) // _RING
        pl.loop(0, _NFULL)(group_round)

        g0 = _NFULL * _RING
        for k, (buf, gs, ss) in enumerate(rings):
            wait_gathers(g0 + k, buf, gs)
            start_store(g0 + k, buf, ss)
        ntail = _NGROUP - (g0 + _RING)
        for k in range(ntail):
            buf, gs, ss = rings[k]
            wait_store(buf, ss)
            start_gathers(g0 + _RING + k, buf, gs)
        for k in range(ntail):
            buf, gs, ss = rings[k]
            wait_gathers(g0 + _RING + k, buf, gs)
            start_store(g0 + _RING + k, buf, ss)
        for buf, gs, ss in rings:
            wait_store(buf, ss)

    return gather_kernel


_gather = _make_gather()


def kernel(input_ids, word_embeddings):
    ids_t = input_ids.astype(jnp.int32).T
    out = _gather(ids_t, word_embeddings)
    return jnp.transpose(out, (1, 0, 2))


# speedup vs baseline: 15.7061x; 1.0009x over previous
import functools

import jax
import jax.numpy as jnp
from jax import lax
from jax.experimental import pallas as pl
from jax.experimental.pallas import tpu as pltpu
from jax.experimental.pallas import tpu_sc as plsc

VOCAB = 100
HIDDEN = 128
BATCH = 4096
HIST = 50

_NC, _NS = 2, 16
_NW = _NC * _NS
_COLS_PW = BATCH // _NW
_TG = 1
_NGROUP = HIST // _TG
_RING = 7


def _make_gather():
    mesh = plsc.VectorSubcoreMesh(core_axis_name="c", subcore_axis_name="s")

    @functools.partial(
        pl.kernel,
        out_type=jax.ShapeDtypeStruct((HIST, BATCH, HIDDEN), jnp.float32),
        mesh=mesh,
        scratch_types=[
            pltpu.VMEM_SHARED((VOCAB, HIDDEN), jnp.float32),
            pltpu.VMEM((HIST, _COLS_PW), jnp.int32),
            *([pltpu.VMEM((_TG, _COLS_PW, HIDDEN), jnp.float32)] * _RING),
            *([pltpu.SemaphoreType.DMA] * (2 * _RING)),
        ],
    )
    def gather_kernel(idx_hbm, table_hbm, out_hbm, table_sh, idx_v, *rest):
        bufs = rest[:_RING]
        gsems = rest[_RING:2 * _RING]
        ssems = rest[2 * _RING:]
        sid = lax.axis_index("s")
        wid = sid * _NC + lax.axis_index("c")
        base = wid * _COLS_PW

        @pl.when(sid == 0)
        def _():
            pltpu.sync_copy(table_hbm, table_sh)

        pltpu.sync_copy(idx_hbm.at[:, pl.ds(base, _COLS_PW)], idx_v)
        plsc.subcore_barrier()

        def start_gathers(g, buf, sem, n=_TG):
            for b in range(n):
                pltpu.async_copy(
                    table_sh.at[idx_v.at[g * _TG + b]], buf.at[b], sem
                )

        def wait_gathers(g, buf, sem, n=_TG):
            for b in range(n):
                pltpu.make_async_copy(
                    table_sh.at[idx_v.at[g * _TG + b]], buf.at[b], sem
                ).wait()

        def start_store(g, buf, sem, n=_TG):
            pltpu.async_copy(
                buf.at[pl.ds(0, n)],
                out_hbm.at[pl.ds(g * _TG, n), pl.ds(base, _COLS_PW)],
                sem,
            )

        def wait_store(buf, sem, n=_TG):
            pltpu.make_async_copy(
                buf.at[pl.ds(0, n)],
                out_hbm.at[pl.ds(0, n), pl.ds(base, _COLS_PW)], sem
            ).wait()

        rings = tuple(zip(bufs, gsems, ssems))

        for k, (buf, gs, _) in enumerate(rings):
            start_gathers(k, buf, gs)

        def group_round(i):
            g = _RING * i
            for k, (buf, gs, ss) in enumerate(rings):
                wait_gathers(g + k, buf, gs)
                start_store(g + k, buf, ss)
            for k, (buf, gs, ss) in enumerate(rings):
                wait_store(buf, ss)
                start_gathers(g + _RING + k, buf, gs)

        _NFULL = (_NGROUP - _RING) // _RING
        pl.loop(0, _NFULL)(group_round)

        g0 = _NFULL * _RING
        for k, (buf, gs, ss) in enumerate(rings):
            wait_gathers(g0 + k, buf, gs)
            start_store(g0 + k, buf, ss)
        ntail = _NGROUP - (g0 + _RING)
        for k in range(ntail):
            buf, gs, ss = rings[k]
            wait_store(buf, ss)
            start_gathers(g0 + _RING + k, buf, gs)
        for k in range(ntail):
            buf, gs, ss = rings[k]
            wait_gathers(g0 + _RING + k, buf, gs)
            start_store(g0 + _RING + k, buf, ss)
        for buf, gs, ss in rings:
            wait_store(buf, ss)

    return gather_kernel


_gather = _make_gather()


def kernel(input_ids, word_embeddings):
    ids_t = input_ids.astype(jnp.int32).T
    out = _gather(ids_t, word_embeddings)
    return jnp.transpose(out, (1, 0, 2))
